# baseline (speedup 1.0000x reference)
import functools

import jax
import jax.numpy as jnp
from jax import lax
from jax.experimental import pallas as pl
from jax.experimental.pallas import tpu as pltpu
from jax.experimental.pallas import tpu_sc as plsc

NUM_IDS = 1000000
FEAT_DIM = 64
OUT_DIM = 128
BATCH = 16384

_RB = 16384
_HK = (NUM_IDS // 2) // _RB
_D = _RB * _HK
_THR = NUM_IDS - _D
_RG = -(-_THR // _RB)
_PAIR_ROWS = _RB * _RG

_NC = 2
_NS = 16
_NW = _NC * _NS
_BPW = BATCH // _NW
_CHUNK = 128
_NCHUNK = _BPW // _CHUNK

_HI = lax.Precision.HIGHEST


def _repack_body(lo_ref, hi_ref, out_ref):
    x2 = jnp.concatenate([lo_ref[...], hi_ref[...]], axis=0)
    i0 = lax.broadcasted_iota(jnp.int32, (2 * FEAT_DIM, 2 * FEAT_DIM), 0)
    i1 = lax.broadcasted_iota(jnp.int32, (2 * FEAT_DIM, 2 * FEAT_DIM), 1)
    eye = (i0 == i1).astype(jnp.float32)
    out_ref[...] = lax.dot_general(x2, eye, (((0,), (0,)), ((), ())))


def _repack(table_t):
    return pl.pallas_call(
        _repack_body,
        grid=(_RG,),
        in_specs=[
            pl.BlockSpec((FEAT_DIM, _RB), lambda i: (0, i)),
            pl.BlockSpec((FEAT_DIM, _RB), lambda i: (0, i + _HK)),
        ],
        out_specs=pl.BlockSpec((_RB, 2 * FEAT_DIM), lambda i: (i, 0)),
        out_shape=jax.ShapeDtypeStruct((_PAIR_ROWS, 2 * FEAT_DIM),
                                       jnp.float32),
    )(table_t, table_t)


def _sc_gather(ids2d, table128):
    mesh = plsc.VectorSubcoreMesh(core_axis_name="c", subcore_axis_name="s")

    @functools.partial(
        pl.kernel,
        mesh=mesh,
        out_type=jax.ShapeDtypeStruct((BATCH, 2 * FEAT_DIM), jnp.float32),
        scratch_types=[
            pltpu.VMEM((_NCHUNK, _CHUNK), jnp.int32),
            pltpu.VMEM((_BPW, 2 * FEAT_DIM), jnp.float32),
            pltpu.SemaphoreType.DMA,
        ],
    )
    def gather_kernel(ids_hbm, table_hbm, out_hbm, idx_v, rows_v, sem):
        wid = lax.axis_index("s") * _NC + lax.axis_index("c")
        pltpu.sync_copy(ids_hbm.at[pl.ds(wid * _NCHUNK, _NCHUNK)], idx_v)
        for j in range(_NCHUNK):
            for i in range(_CHUNK // 16):
                sl = (j, pl.ds(i * 16, 16))
                v = idx_v[sl]
                idx_v[sl] = jnp.where(v >= _THR, v - _D, v)
        copies = [
            pltpu.async_copy(
                table_hbm.at[idx_v.at[j]],
                rows_v.at[pl.ds(j * _CHUNK, _CHUNK)],
                sem,
            )
            for j in range(_NCHUNK)
        ]
        for cp in copies:
            cp.wait()
        pltpu.sync_copy(rows_v, out_hbm.at[pl.ds(wid * _BPW, _BPW)])

    return gather_kernel(ids2d, table128)


def _ln(x):
    mu = jnp.mean(x, axis=-1, keepdims=True)
    var = jnp.mean((x - mu) * (x - mu), axis=-1, keepdims=True)
    return (x - mu) * lax.rsqrt(var + 1e-5)


def _gelu(x):
    return x * 0.5 * (1.0 + lax.erf(x * 0.7071067811865476))


_BB = 1024
_NB = BATCH // _BB


def _mlp_body(pair_ref, ids_ref, g_ref, a_ref, o_ref, z_ref, w1d_ref,
              w1b_ref, b1_ref, w2_ref, b2_ref, w3_ref, b3_ref, out_ref):
    g = g_ref[0]
    a = a_ref[0]
    o = o_ref[0]
    iota = lax.broadcasted_iota(jnp.int32, (32, _BB), 0)
    tgt = jnp.where(iota < 2, g, jnp.where(iota < 9, a + 2, o + 9))
    mh = (iota == tgt).astype(jnp.float32)
    p = lax.dot_general(z_ref[...], w1b_ref[...], (((1,), (0,)), ((), ())),
                        precision=_HI)
    hc = lax.dot_general(mh, p, (((0,), (0,)), ((), ())), precision=_HI)
    pr = (ids_ref[0] >= _THR).astype(jnp.float32)
    pm = lax.dot_general(pr, jnp.ones((1, 2 * FEAT_DIM), jnp.float32),
                         (((0,), (0,)), ((), ())), precision=_HI)
    li = lax.broadcasted_iota(jnp.int32, (_BB, 2 * FEAT_DIM), 1)
    m = jnp.where(li < FEAT_DIM, 1.0 - pm, pm)
    h = lax.dot_general(pair_ref[...] * m, w1d_ref[...],
                        (((1,), (0,)), ((), ())),
                        precision=_HI) + hc + b1_ref[...]
    h = _gelu(_ln(h))
    h = lax.dot_general(h, w2_ref[...], (((1,), (0,)), ((), ())),
                        precision=_HI) + b2_ref[...]
    h = _gelu(_ln(h))
    h = lax.dot_general(h, w3_ref[...], (((1,), (0,)), ((), ())),
                        precision=_HI) + b3_ref[...]
    out_ref[...] = _gelu(h)


def _mlp(pair_emb, ids3, g3, a3, o3, z, w1d, w1b, b1r, w2, b2r, w3, b3r,
         interpret=False):
    full = lambda shape: pl.BlockSpec(shape, lambda i: (0,) * len(shape))
    return pl.pallas_call(
        _mlp_body,
        grid=(_NB,),
        in_specs=[
            pl.BlockSpec((_BB, 2 * FEAT_DIM), lambda i: (i, 0)),
            pl.BlockSpec((1, 1, _BB), lambda i: (i, 0, 0)),
            pl.BlockSpec((1, 1, _BB), lambda i: (i, 0, 0)),
            pl.BlockSpec((1, 1, _BB), lambda i: (i, 0, 0)),
            pl.BlockSpec((1, 1, _BB), lambda i: (i, 0, 0)),
            full((32, 3 * FEAT_DIM)),
            full((2 * FEAT_DIM, 128)),
            full((3 * FEAT_DIM, 128)),
            full((1, 128)),
            full((128, 64)),
            full((1, 64)),
            full((64, OUT_DIM)),
            full((1, OUT_DIM)),
        ],
        out_specs=pl.BlockSpec((_BB, OUT_DIM), lambda i: (i, 0)),
        out_shape=jax.ShapeDtypeStruct((BATCH, OUT_DIM), jnp.float32),
        interpret=interpret,
    )(pair_emb, ids3, g3, a3, o3, z, w1d, w1b, b1r, w2, b2r, w3, b3r)


def kernel(user_ids, user_genders, user_ages, user_occs, id_table,
           gender_table, age_table, occ_table, W1, b1, W2, b2, W3, b3):
    table128 = _repack(id_table.T)

    ids2d = user_ids.reshape(_NW * _NCHUNK, _CHUNK)
    pair_emb = _sc_gather(ids2d, table128)

    z = jnp.zeros((32, 3 * FEAT_DIM), jnp.float32)
    z = z.at[0:2, 0:FEAT_DIM].set(gender_table)
    z = z.at[2:9, FEAT_DIM:2 * FEAT_DIM].set(age_table)
    z = z.at[9:30, 2 * FEAT_DIM:3 * FEAT_DIM].set(occ_table)

    ids3 = user_ids.reshape(_NB, 1, _BB)
    g3 = user_genders.reshape(_NB, 1, _BB)
    a3 = user_ages.reshape(_NB, 1, _BB)
    o3 = user_occs.reshape(_NB, 1, _BB)

    w1d = jnp.concatenate([W1[:FEAT_DIM], W1[:FEAT_DIM]], axis=0)
    return _mlp(pair_emb, ids3, g3, a3, o3, z,
                w1d, W1[FEAT_DIM:],
                b1.reshape(1, -1), W2, b2.reshape(1, -1),
                W3, b3.reshape(1, -1))


# speedup vs baseline: 1.0297x; 1.0297x over previous
import functools

import jax
import jax.numpy as jnp
from jax import lax
from jax.experimental import pallas as pl
from jax.experimental.pallas import tpu as pltpu
from jax.experimental.pallas import tpu_sc as plsc

NUM_IDS = 1000000
FEAT_DIM = 64
OUT_DIM = 128
BATCH = 16384

_RB = 16384
_HK = (NUM_IDS // 2) // _RB
_D = _RB * _HK
_THR = NUM_IDS - _D
_RG = -(-_THR // _RB)
_PAIR_ROWS = _RB * _RG

_NC = 2
_NS = 16
_NW = _NC * _NS
_BPW = BATCH // _NW
_CHUNK = 128
_NCHUNK = _BPW // _CHUNK

_HI = lax.Precision.HIGHEST


def _repack_body(lo_ref, hi_ref, out_ref):
    x2 = jnp.concatenate([lo_ref[...], hi_ref[...]], axis=0)
    i0 = lax.broadcasted_iota(jnp.int32, (2 * FEAT_DIM, 2 * FEAT_DIM), 0)
    i1 = lax.broadcasted_iota(jnp.int32, (2 * FEAT_DIM, 2 * FEAT_DIM), 1)
    eye = (i0 == i1).astype(jnp.float32)
    out_ref[...] = lax.dot_general(x2, eye, (((0,), (0,)), ((), ())))


def _repack(table_t):
    return pl.pallas_call(
        _repack_body,
        grid=(_RG,),
        in_specs=[
            pl.BlockSpec((FEAT_DIM, _RB), lambda i: (0, i)),
            pl.BlockSpec((FEAT_DIM, _RB), lambda i: (0, i + _HK)),
        ],
        out_specs=pl.BlockSpec((_RB, 2 * FEAT_DIM), lambda i: (i, 0)),
        out_shape=jax.ShapeDtypeStruct((_PAIR_ROWS, 2 * FEAT_DIM),
                                       jnp.float32),
    )(table_t, table_t)


def _sc_gather(ids2d, table128):
    mesh = plsc.VectorSubcoreMesh(core_axis_name="c", subcore_axis_name="s")

    @functools.partial(
        pl.kernel,
        mesh=mesh,
        out_type=jax.ShapeDtypeStruct((BATCH, 2 * FEAT_DIM), jnp.float32),
        scratch_types=[
            pltpu.VMEM((_NCHUNK, _CHUNK), jnp.int32),
            pltpu.VMEM((_BPW, 2 * FEAT_DIM), jnp.float32),
            pltpu.SemaphoreType.DMA,
        ],
    )
    def gather_kernel(ids_hbm, table_hbm, out_hbm, idx_v, rows_v, sem):
        wid = lax.axis_index("s") * _NC + lax.axis_index("c")
        pltpu.sync_copy(ids_hbm.at[pl.ds(wid * _NCHUNK, _NCHUNK)], idx_v)
        for j in range(_NCHUNK):
            for i in range(_CHUNK // 16):
                sl = (j, pl.ds(i * 16, 16))
                v = idx_v[sl]
                idx_v[sl] = jnp.where(v >= _THR, v - _D, v)
        copies = [
            pltpu.async_copy(
                table_hbm.at[idx_v.at[j]],
                rows_v.at[pl.ds(j * _CHUNK, _CHUNK)],
                sem,
            )
            for j in range(_NCHUNK)
        ]
        for cp in copies:
            cp.wait()
        pltpu.sync_copy(rows_v, out_hbm.at[pl.ds(wid * _BPW, _BPW)])

    return gather_kernel(ids2d, table128)


def _ln(x):
    mu = jnp.mean(x, axis=-1, keepdims=True)
    var = jnp.mean((x - mu) * (x - mu), axis=-1, keepdims=True)
    return (x - mu) * lax.rsqrt(var + 1e-5)


def _gelu(x):
    return x * 0.5 * (1.0 + lax.erf(x * 0.7071067811865476))


_BB = 2048
_NB = BATCH // _BB


def _mlp_body(pair_ref, ids_ref, g_ref, a_ref, o_ref, z_ref, w1d_ref,
              w1b_ref, b1_ref, w2_ref, b2_ref, w3_ref, b3_ref, out_ref):
    g = g_ref[0]
    a = a_ref[0]
    o = o_ref[0]
    iota = lax.broadcasted_iota(jnp.int32, (32, _BB), 0)
    tgt = jnp.where(iota < 2, g, jnp.where(iota < 9, a + 2, o + 9))
    mh = (iota == tgt).astype(jnp.float32)
    p = lax.dot_general(z_ref[...], w1b_ref[...], (((1,), (0,)), ((), ())),
                        precision=_HI)
    hc = lax.dot_general(mh, p, (((0,), (0,)), ((), ())), precision=_HI)
    pr = (ids_ref[0] >= _THR).astype(jnp.float32)
    pm = lax.dot_general(pr, jnp.ones((1, 2 * FEAT_DIM), jnp.float32),
                         (((0,), (0,)), ((), ())), precision=_HI)
    li = lax.broadcasted_iota(jnp.int32, (_BB, 2 * FEAT_DIM), 1)
    m = jnp.where(li < FEAT_DIM, 1.0 - pm, pm)
    h = lax.dot_general(pair_ref[...] * m, w1d_ref[...],
                        (((1,), (0,)), ((), ())),
                        precision=_HI) + hc + b1_ref[...]
    h = _gelu(_ln(h))
    h = lax.dot_general(h, w2_ref[...], (((1,), (0,)), ((), ())),
                        precision=_HI) + b2_ref[...]
    h = _gelu(_ln(h))
    h = lax.dot_general(h, w3_ref[...], (((1,), (0,)), ((), ())),
                        precision=_HI) + b3_ref[...]
    out_ref[...] = _gelu(h)


def _mlp(pair_emb, ids3, g3, a3, o3, z, w1d, w1b, b1r, w2, b2r, w3, b3r,
         interpret=False):
    full = lambda shape: pl.BlockSpec(shape, lambda i: (0,) * len(shape))
    return pl.pallas_call(
        _mlp_body,
        grid=(_NB,),
        in_specs=[
            pl.BlockSpec((_BB, 2 * FEAT_DIM), lambda i: (i, 0)),
            pl.BlockSpec((1, 1, _BB), lambda i: (i, 0, 0)),
            pl.BlockSpec((1, 1, _BB), lambda i: (i, 0, 0)),
            pl.BlockSpec((1, 1, _BB), lambda i: (i, 0, 0)),
            pl.BlockSpec((1, 1, _BB), lambda i: (i, 0, 0)),
            full((32, 3 * FEAT_DIM)),
            full((2 * FEAT_DIM, 128)),
            full((3 * FEAT_DIM, 128)),
            full((1, 128)),
            full((128, 64)),
            full((1, 64)),
            full((64, OUT_DIM)),
            full((1, OUT_DIM)),
        ],
        out_specs=pl.BlockSpec((_BB, OUT_DIM), lambda i: (i, 0)),
        out_shape=jax.ShapeDtypeStruct((BATCH, OUT_DIM), jnp.float32),
        interpret=interpret,
    )(pair_emb, ids3, g3, a3, o3, z, w1d, w1b, b1r, w2, b2r, w3, b3r)


def kernel(user_ids, user_genders, user_ages, user_occs, id_table,
           gender_table, age_table, occ_table, W1, b1, W2, b2, W3, b3):
    table128 = _repack(id_table.T)

    ids2d = user_ids.reshape(_NW * _NCHUNK, _CHUNK)
    pair_emb = _sc_gather(ids2d, table128)

    z = jnp.zeros((32, 3 * FEAT_DIM), jnp.float32)
    z = z.at[0:2, 0:FEAT_DIM].set(gender_table)
    z = z.at[2:9, FEAT_DIM:2 * FEAT_DIM].set(age_table)
    z = z.at[9:30, 2 * FEAT_DIM:3 * FEAT_DIM].set(occ_table)

    ids3 = user_ids.reshape(_NB, 1, _BB)
    g3 = user_genders.reshape(_NB, 1, _BB)
    a3 = user_ages.reshape(_NB, 1, _BB)
    o3 = user_occs.reshape(_NB, 1, _BB)

    w1d = jnp.concatenate([W1[:FEAT_DIM], W1[:FEAT_DIM]], axis=0)
    return _mlp(pair_emb, ids3, g3, a3, o3, z,
                w1d, W1[FEAT_DIM:],
                b1.reshape(1, -1), W2, b2.reshape(1, -1),
                W3, b3.reshape(1, -1))
